# baseline (speedup 1.0000x reference)
import functools

import jax
import jax.numpy as jnp
from jax import lax
from jax.experimental import pallas as pl
from jax.experimental.pallas import tpu as pltpu
from jax.experimental.pallas import tpu_sc as plsc

N = 10000
E = 320000
H = 128
NC, NS = 2, 16
NW = NC * NS
CHUNK = 128
CPT = 80
EPAD = NW * CPT * CHUNK
R = 10240
ZROWS = R // NS

_mesh = plsc.VectorSubcoreMesh(core_axis_name="c", subcore_axis_name="s")


@functools.partial(
    pl.kernel,
    out_type=jax.ShapeDtypeStruct((NC, R, H), jnp.float32),
    mesh=_mesh,
    scratch_types=[
        pltpu.VMEM((CPT, CHUNK), jnp.int32),
        pltpu.VMEM((CHUNK, H), jnp.float32),
        pltpu.VMEM_SHARED((R, H), jnp.float32),
    ],
)
def _sc_degree(dstb_hbm, ones_hbm, out_hbm, dst_v, ones_v, accum):
    c = lax.axis_index("c")
    s = lax.axis_index("s")
    w = s * NC + c

    def zrow(i, carry):
        for k in range(H // 16):
            ones_v[i, pl.ds(k * 16, 16)] = jnp.zeros((16,), jnp.float32)
        return carry

    lax.fori_loop(0, CHUNK, zrow, 0)
    for q in range(ZROWS // CHUNK):
        pltpu.sync_copy(ones_v,
                        accum.at[pl.ds(s * ZROWS + q * CHUNK, CHUNK)])
    pltpu.sync_copy(ones_hbm, ones_v)
    pltpu.sync_copy(dstb_hbm.at[pl.ds(w * CPT, CPT)], dst_v)
    plsc.subcore_barrier()

    def body(j, carry):
        pltpu.sync_copy(ones_v, accum.at[dst_v.at[j]], add=True)
        return carry

    lax.fori_loop(0, CPT, body, 0)
    plsc.subcore_barrier()
    pltpu.sync_copy(accum.at[pl.ds(s * ZROWS, ZROWS)],
                    out_hbm.at[c].at[pl.ds(s * ZROWS, ZROWS)])


NB = 2
SLOW_C = 1
SLOW_CPT = 0
FAST_CPT = 160
PAIR = SLOW_CPT + FAST_CPT
NSTAGE = 5
SLOW_PER_STAGE = SLOW_CPT // NSTAGE
FAST_PER_STAGE = FAST_CPT // NSTAGE
QC = 32


@functools.partial(
    pl.kernel,
    out_type=jax.ShapeDtypeStruct((NC, R, H), jnp.float32),
    mesh=_mesh,
    scratch_types=[
        pltpu.VMEM((QC, CHUNK), jnp.int32),
        pltpu.VMEM((QC, CHUNK), jnp.int32),
        pltpu.VMEM((CHUNK, H), jnp.float32),
        pltpu.VMEM((CHUNK, H), jnp.float32),
        pltpu.VMEM_SHARED((R, H), jnp.float32),
        pltpu.SemaphoreType.DMA,
        pltpu.SemaphoreType.DMA,
    ],
)
def _sc_scatter(h_hbm, srcb_hbm, dstb_hbm, out_hbm,
                src_v, dst_v, r0, r1, accum, s0, s1):
    rows = (r0, r1)
    sems = (s0, s1)
    c = lax.axis_index("c")
    s = lax.axis_index("s")

    def zrow(i, carry):
        for k in range(H // 16):
            r0[i, pl.ds(k * 16, 16)] = jnp.zeros((16,), jnp.float32)
        return carry

    lax.fori_loop(0, CHUNK, zrow, 0)
    for q in range(ZROWS // CHUNK):
        pltpu.sync_copy(r0, accum.at[pl.ds(s * ZROWS + q * CHUNK, CHUNK)])
    plsc.subcore_barrier()

    def _run(base_w, per_stage):
        if per_stage == 0:
            return
        for st in range(NSTAGE):
            sbase = pl.multiple_of(base_w + st * per_stage, 8)
            pltpu.sync_copy(srcb_hbm.at[pl.ds(sbase, QC)], src_v)
            pltpu.sync_copy(dstb_hbm.at[pl.ds(sbase, QC)], dst_v)

            def body(i, carry2):
                j0 = i * NB
                for b in range(NB):
                    pltpu.async_copy(h_hbm.at[src_v.at[j0 + b]], rows[b],
                                     sems[b])
                for b in range(NB):
                    pltpu.make_async_copy(h_hbm.at[src_v.at[j0 + b]],
                                          rows[b], sems[b]).wait()
                    pltpu.sync_copy(rows[b], accum.at[dst_v.at[j0 + b]],
                                    add=True)
                return carry2

            lax.fori_loop(0, per_stage // NB, body, 0)

    @pl.when(c == SLOW_C)
    def _():
        _run(s * PAIR + FAST_CPT, SLOW_PER_STAGE)

    @pl.when(c != SLOW_C)
    def _():
        _run(s * PAIR, FAST_PER_STAGE)

    plsc.subcore_barrier()
    pltpu.sync_copy(accum.at[pl.ds(s * ZROWS, ZROWS)],
                    out_hbm.at[c].at[pl.ds(s * ZROWS, ZROWS)])


def _tcmm_body(x_ref, w_ref, h_ref):
    h_ref[...] = jnp.dot(x_ref[...], w_ref[...],
                         preferred_element_type=jnp.float32)


def _tcmm(x, w1):
    return pl.pallas_call(
        _tcmm_body,
        out_shape=jax.ShapeDtypeStruct((N, H), jnp.float32),
    )(x, w1)


def _tc1_body(h_ref, d0_ref, d1_ref, hp_ref, dinv_ref):
    deg = d0_ref[...] + d1_ref[...] + 1.0
    dinv = lax.rsqrt(deg)
    dinv_ref[...] = dinv
    hp_ref[...] = h_ref[...] * dinv


def _tc1(h1, d0, d1):
    return pl.pallas_call(
        _tc1_body,
        out_shape=[jax.ShapeDtypeStruct((N, H), jnp.float32),
                   jax.ShapeDtypeStruct((N, 1), jnp.float32)],
    )(h1, d0, d1)


def _tcmid_body(a0_ref, a1_ref, hp_ref, dinv_ref, b_ref, g_ref, be_ref,
                wn_ref, out_ref):
    dinv = dinv_ref[...]
    z = jnp.maximum(dinv * (a0_ref[...] + a1_ref[...] + hp_ref[...])
                    + b_ref[...], 0.0)
    m = jnp.mean(z, axis=0, keepdims=True)
    v = jnp.mean((z - m) ** 2, axis=0, keepdims=True)
    zn = (z - m) * lax.rsqrt(v + 1e-5) * g_ref[...] + be_ref[...]
    out_ref[...] = jnp.dot(zn, wn_ref[...],
                           preferred_element_type=jnp.float32) * dinv


def _tcmid(a0, a1, hp, dinv, b, g, be, wn):
    return pl.pallas_call(
        _tcmid_body,
        out_shape=jax.ShapeDtypeStruct((N, H), jnp.float32),
    )(a0, a1, hp, dinv, b, g, be, wn)


def _tcfin_body(a0_ref, a1_ref, hp_ref, dinv_ref, b_ref, g_ref, be_ref,
                wc_ref, bc_ref, out_ref):
    dinv = dinv_ref[...]
    z = jnp.maximum(dinv * (a0_ref[...] + a1_ref[...] + hp_ref[...])
                    + b_ref[...], 0.0)
    m = jnp.mean(z, axis=0, keepdims=True)
    v = jnp.mean((z - m) ** 2, axis=0, keepdims=True)
    zn = (z - m) * lax.rsqrt(v + 1e-5) * g_ref[...] + be_ref[...]
    out_ref[...] = jnp.dot(zn, wc_ref[...],
                           preferred_element_type=jnp.float32) + bc_ref[...]


def _tcfin(a0, a1, hp, dinv, b, g, be, wc, bc):
    return pl.pallas_call(
        _tcfin_body,
        out_shape=jax.ShapeDtypeStruct((N, 2), jnp.float32),
    )(a0, a1, hp, dinv, b, g, be, wc, bc)


def kernel(x, edge_index, W1, b1, W2, b2, W3, b3,
           g1, be1, g2, be2, g3, be3, Wc, bc):
    src = edge_index[0]
    dst = edge_index[1]
    padlen = EPAD - E
    srcb = jnp.concatenate(
        [src, jnp.zeros((padlen,), src.dtype)]).reshape(NW * CPT, CHUNK)
    pad_dst = N + jnp.arange(padlen, dtype=dst.dtype) % (R - N)
    dstb = jnp.concatenate([dst, pad_dst]).reshape(NW * CPT, CHUNK)
    srcb = jnp.pad(srcb, ((0, 32), (0, 0)))
    dstb = jnp.pad(dstb, ((0, 32), (0, 0)))
    ones_deg = jnp.ones((CHUNK, H), jnp.float32)

    h1 = _tcmm(x, W1)
    degp = _sc_degree(dstb, ones_deg)
    d0 = degp[0, :N, 0:1]
    d1 = degp[1, :N, 0:1]
    hp, dinv = _tc1(h1, d0, d1)

    layer_params = [(b1, g1, be1, W2), (b2, g2, be2, W3)]
    for b, g, be, wn in layer_params:
        agg = _sc_scatter(hp, srcb, dstb)
        hp = _tcmid(agg[0, :N], agg[1, :N], hp, dinv,
                    b.reshape(1, H), g.reshape(1, H), be.reshape(1, H), wn)

    agg = _sc_scatter(hp, srcb, dstb)
    out = _tcfin(agg[0, :N], agg[1, :N], hp, dinv,
                 b3.reshape(1, H), g3.reshape(1, H), be3.reshape(1, H),
                 Wc, bc.reshape(1, 2))
    return out


# speedup vs baseline: 1.4470x; 1.4470x over previous
import functools

import jax
import jax.numpy as jnp
from jax import lax
from jax.experimental import pallas as pl
from jax.experimental.pallas import tpu as pltpu
from jax.experimental.pallas import tpu_sc as plsc

N = 10000
E = 320000
H = 128
NC, NS = 2, 16
NW = NC * NS
CHUNK = 128
CPT = 80
EPAD = NW * CPT * CHUNK
R = 10240
ZROWS = R // NS

_mesh = plsc.VectorSubcoreMesh(core_axis_name="c", subcore_axis_name="s")


@functools.partial(
    pl.kernel,
    out_type=jax.ShapeDtypeStruct((NC, R, H), jnp.float32),
    mesh=_mesh,
    scratch_types=[
        pltpu.VMEM((CPT, CHUNK), jnp.int32),
        pltpu.VMEM((CHUNK, H), jnp.float32),
        pltpu.VMEM_SHARED((R, H), jnp.float32),
    ],
)
def _sc_degree(dstb_hbm, ones_hbm, out_hbm, dst_v, ones_v, accum):
    c = lax.axis_index("c")
    s = lax.axis_index("s")
    w = s * NC + c

    def zrow(i, carry):
        for k in range(H // 16):
            ones_v[i, pl.ds(k * 16, 16)] = jnp.zeros((16,), jnp.float32)
        return carry

    lax.fori_loop(0, CHUNK, zrow, 0)
    for q in range(ZROWS // CHUNK):
        pltpu.sync_copy(ones_v,
                        accum.at[pl.ds(s * ZROWS + q * CHUNK, CHUNK)])
    pltpu.sync_copy(ones_hbm, ones_v)
    pltpu.sync_copy(dstb_hbm.at[pl.ds(w * CPT, CPT)], dst_v)
    plsc.subcore_barrier()

    def body(j, carry):
        pltpu.sync_copy(ones_v, accum.at[dst_v.at[j]], add=True)
        return carry

    lax.fori_loop(0, CPT, body, 0)
    plsc.subcore_barrier()
    pltpu.sync_copy(accum.at[pl.ds(s * ZROWS, ZROWS)],
                    out_hbm.at[c].at[pl.ds(s * ZROWS, ZROWS)])


NB = 2
SLOW_C = 1
SLOW_CPT = 40
FAST_CPT = 120
PAIR = SLOW_CPT + FAST_CPT
QC = 40
SLOW_NST = SLOW_CPT // QC
FAST_NST = FAST_CPT // QC


@functools.partial(
    pl.kernel,
    out_type=jax.ShapeDtypeStruct((NC, R, H), jnp.float32),
    mesh=_mesh,
    scratch_types=[
        pltpu.VMEM((QC, CHUNK), jnp.int32),
        pltpu.VMEM((QC, CHUNK), jnp.int32),
        pltpu.VMEM((CHUNK, H), jnp.float32),
        pltpu.VMEM((CHUNK, H), jnp.float32),
        pltpu.VMEM_SHARED((R, H), jnp.float32),
        pltpu.SemaphoreType.DMA,
        pltpu.SemaphoreType.DMA,
    ],
)
def _sc_scatter(h_hbm, srcb_hbm, dstb_hbm, out_hbm,
                src_v, dst_v, r0, r1, accum, s0, s1):
    rows = (r0, r1)
    sems = (s0, s1)
    c = lax.axis_index("c")
    s = lax.axis_index("s")

    def zrow(i, carry):
        for k in range(H // 16):
            r0[i, pl.ds(k * 16, 16)] = jnp.zeros((16,), jnp.float32)
        return carry

    lax.fori_loop(0, CHUNK, zrow, 0)
    for q in range(ZROWS // CHUNK):
        pltpu.sync_copy(r0, accum.at[pl.ds(s * ZROWS + q * CHUNK, CHUNK)])
    plsc.subcore_barrier()

    def _run(base_w, nstages):
        for st in range(nstages):
            sbase = pl.multiple_of(base_w + st * QC, 8)
            pltpu.sync_copy(srcb_hbm.at[pl.ds(sbase, QC)], src_v)
            pltpu.sync_copy(dstb_hbm.at[pl.ds(sbase, QC)], dst_v)

            def body(i, carry2):
                j0 = i * NB
                for b in range(NB):
                    pltpu.async_copy(h_hbm.at[src_v.at[j0 + b]], rows[b],
                                     sems[b])
                for b in range(NB):
                    pltpu.make_async_copy(h_hbm.at[src_v.at[j0 + b]],
                                          rows[b], sems[b]).wait()
                    pltpu.sync_copy(rows[b], accum.at[dst_v.at[j0 + b]],
                                    add=True)
                return carry2

            lax.fori_loop(0, QC // NB, body, 0)

    @pl.when(c == SLOW_C)
    def _():
        _run(s * PAIR + FAST_CPT, SLOW_NST)

    @pl.when(c != SLOW_C)
    def _():
        _run(s * PAIR, FAST_NST)

    plsc.subcore_barrier()
    pltpu.sync_copy(accum.at[pl.ds(s * ZROWS, ZROWS)],
                    out_hbm.at[c].at[pl.ds(s * ZROWS, ZROWS)])


def _tcmm_body(x_ref, w_ref, h_ref):
    h_ref[...] = jnp.dot(x_ref[...], w_ref[...],
                         preferred_element_type=jnp.float32)


def _tcmm(x, w1):
    return pl.pallas_call(
        _tcmm_body,
        out_shape=jax.ShapeDtypeStruct((N, H), jnp.float32),
    )(x, w1)


def _tc1_body(h_ref, d0_ref, d1_ref, hp_ref, dinv_ref):
    deg = d0_ref[...] + d1_ref[...] + 1.0
    dinv = lax.rsqrt(deg)
    dinv_ref[...] = dinv
    hp_ref[...] = h_ref[...] * dinv


def _tc1(h1, d0, d1):
    return pl.pallas_call(
        _tc1_body,
        out_shape=[jax.ShapeDtypeStruct((N, H), jnp.float32),
                   jax.ShapeDtypeStruct((N, 1), jnp.float32)],
    )(h1, d0, d1)


def _tcmid_body(a0_ref, a1_ref, hp_ref, dinv_ref, b_ref, g_ref, be_ref,
                wn_ref, out_ref):
    dinv = dinv_ref[...]
    z = jnp.maximum(dinv * (a0_ref[...] + a1_ref[...] + hp_ref[...])
                    + b_ref[...], 0.0)
    m = jnp.mean(z, axis=0, keepdims=True)
    v = jnp.mean((z - m) ** 2, axis=0, keepdims=True)
    zn = (z - m) * lax.rsqrt(v + 1e-5) * g_ref[...] + be_ref[...]
    out_ref[...] = jnp.dot(zn, wn_ref[...],
                           preferred_element_type=jnp.float32) * dinv


def _tcmid(a0, a1, hp, dinv, b, g, be, wn):
    return pl.pallas_call(
        _tcmid_body,
        out_shape=jax.ShapeDtypeStruct((N, H), jnp.float32),
    )(a0, a1, hp, dinv, b, g, be, wn)


def _tcfin_body(a0_ref, a1_ref, hp_ref, dinv_ref, b_ref, g_ref, be_ref,
                wc_ref, bc_ref, out_ref):
    dinv = dinv_ref[...]
    z = jnp.maximum(dinv * (a0_ref[...] + a1_ref[...] + hp_ref[...])
                    + b_ref[...], 0.0)
    m = jnp.mean(z, axis=0, keepdims=True)
    v = jnp.mean((z - m) ** 2, axis=0, keepdims=True)
    zn = (z - m) * lax.rsqrt(v + 1e-5) * g_ref[...] + be_ref[...]
    out_ref[...] = jnp.dot(zn, wc_ref[...],
                           preferred_element_type=jnp.float32) + bc_ref[...]


def _tcfin(a0, a1, hp, dinv, b, g, be, wc, bc):
    return pl.pallas_call(
        _tcfin_body,
        out_shape=jax.ShapeDtypeStruct((N, 2), jnp.float32),
    )(a0, a1, hp, dinv, b, g, be, wc, bc)


def kernel(x, edge_index, W1, b1, W2, b2, W3, b3,
           g1, be1, g2, be2, g3, be3, Wc, bc):
    src = edge_index[0]
    dst = edge_index[1]
    padlen = EPAD - E
    srcb = jnp.concatenate(
        [src, jnp.zeros((padlen,), src.dtype)]).reshape(NW * CPT, CHUNK)
    pad_dst = N + jnp.arange(padlen, dtype=dst.dtype) % (R - N)
    dstb = jnp.concatenate([dst, pad_dst]).reshape(NW * CPT, CHUNK)
    srcb = jnp.pad(srcb, ((0, 32), (0, 0)))
    dstb = jnp.pad(dstb, ((0, 32), (0, 0)))
    ones_deg = jnp.ones((CHUNK, H), jnp.float32)

    h1 = _tcmm(x, W1)
    degp = _sc_degree(dstb, ones_deg)
    d0 = degp[0, :N, 0:1]
    d1 = degp[1, :N, 0:1]
    hp, dinv = _tc1(h1, d0, d1)

    layer_params = [(b1, g1, be1, W2), (b2, g2, be2, W3)]
    for b, g, be, wn in layer_params:
        agg = _sc_scatter(hp, srcb, dstb)
        hp = _tcmid(agg[0, :N], agg[1, :N], hp, dinv,
                    b.reshape(1, H), g.reshape(1, H), be.reshape(1, H), wn)

    agg = _sc_scatter(hp, srcb, dstb)
    out = _tcfin(agg[0, :N], agg[1, :N], hp, dinv,
                 b3.reshape(1, H), g3.reshape(1, H), be3.reshape(1, H),
                 Wc, bc.reshape(1, 2))
    return out


# speedup vs baseline: 1.4488x; 1.0013x over previous
import functools

import jax
import jax.numpy as jnp
from jax import lax
from jax.experimental import pallas as pl
from jax.experimental.pallas import tpu as pltpu
from jax.experimental.pallas import tpu_sc as plsc

N = 10000
E = 320000
H = 128
NC, NS = 2, 16
NW = NC * NS
CHUNK = 128
CPT = 80
EPAD = NW * CPT * CHUNK
R = 10240
ZROWS = R // NS

_mesh = plsc.VectorSubcoreMesh(core_axis_name="c", subcore_axis_name="s")


@functools.partial(
    pl.kernel,
    out_type=jax.ShapeDtypeStruct((NC, R, H), jnp.float32),
    mesh=_mesh,
    scratch_types=[
        pltpu.VMEM((CPT, CHUNK), jnp.int32),
        pltpu.VMEM((CHUNK, H), jnp.float32),
        pltpu.VMEM_SHARED((R, H), jnp.float32),
    ],
)
def _sc_degree(dstb_hbm, ones_hbm, out_hbm, dst_v, ones_v, accum):
    c = lax.axis_index("c")
    s = lax.axis_index("s")
    w = s * NC + c

    def zrow(i, carry):
        for k in range(H // 16):
            ones_v[i, pl.ds(k * 16, 16)] = jnp.zeros((16,), jnp.float32)
        return carry

    lax.fori_loop(0, CHUNK, zrow, 0)
    for q in range(ZROWS // CHUNK):
        pltpu.sync_copy(ones_v,
                        accum.at[pl.ds(s * ZROWS + q * CHUNK, CHUNK)])
    pltpu.sync_copy(ones_hbm, ones_v)
    pltpu.sync_copy(dstb_hbm.at[pl.ds(w * CPT, CPT)], dst_v)
    plsc.subcore_barrier()

    def body(j, carry):
        pltpu.sync_copy(ones_v, accum.at[dst_v.at[j]], add=True)
        return carry

    lax.fori_loop(0, CPT, body, 0)
    plsc.subcore_barrier()
    pltpu.sync_copy(accum.at[pl.ds(s * ZROWS, ZROWS)],
                    out_hbm.at[c].at[pl.ds(s * ZROWS, ZROWS)])


NB = 2
SLOW_C = 0
SLOW_CPT = 40
FAST_CPT = 120
PAIR = SLOW_CPT + FAST_CPT
QC = 40
SLOW_NST = SLOW_CPT // QC
FAST_NST = FAST_CPT // QC


@functools.partial(
    pl.kernel,
    out_type=jax.ShapeDtypeStruct((NC, R, H), jnp.float32),
    mesh=_mesh,
    scratch_types=[
        pltpu.VMEM((QC, CHUNK), jnp.int32),
        pltpu.VMEM((QC, CHUNK), jnp.int32),
        pltpu.VMEM((CHUNK, H), jnp.float32),
        pltpu.VMEM((CHUNK, H), jnp.float32),
        pltpu.VMEM_SHARED((R, H), jnp.float32),
        pltpu.SemaphoreType.DMA,
        pltpu.SemaphoreType.DMA,
    ],
)
def _sc_scatter(h_hbm, srcb_hbm, dstb_hbm, out_hbm,
                src_v, dst_v, r0, r1, accum, s0, s1):
    rows = (r0, r1)
    sems = (s0, s1)
    c = lax.axis_index("c")
    s = lax.axis_index("s")

    def zrow(i, carry):
        for k in range(H // 16):
            r0[i, pl.ds(k * 16, 16)] = jnp.zeros((16,), jnp.float32)
        return carry

    lax.fori_loop(0, CHUNK, zrow, 0)
    for q in range(ZROWS // CHUNK):
        pltpu.sync_copy(r0, accum.at[pl.ds(s * ZROWS + q * CHUNK, CHUNK)])
    plsc.subcore_barrier()

    def _run(base_w, nstages):
        for st in range(nstages):
            sbase = pl.multiple_of(base_w + st * QC, 8)
            pltpu.sync_copy(srcb_hbm.at[pl.ds(sbase, QC)], src_v)
            pltpu.sync_copy(dstb_hbm.at[pl.ds(sbase, QC)], dst_v)

            def body(i, carry2):
                j0 = i * NB
                for b in range(NB):
                    pltpu.async_copy(h_hbm.at[src_v.at[j0 + b]], rows[b],
                                     sems[b])
                for b in range(NB):
                    pltpu.make_async_copy(h_hbm.at[src_v.at[j0 + b]],
                                          rows[b], sems[b]).wait()
                    pltpu.sync_copy(rows[b], accum.at[dst_v.at[j0 + b]],
                                    add=True)
                return carry2

            lax.fori_loop(0, QC // NB, body, 0)

    @pl.when(c == SLOW_C)
    def _():
        _run(s * PAIR + FAST_CPT, SLOW_NST)

    @pl.when(c != SLOW_C)
    def _():
        _run(s * PAIR, FAST_NST)

    plsc.subcore_barrier()
    pltpu.sync_copy(accum.at[pl.ds(s * ZROWS, ZROWS)],
                    out_hbm.at[c].at[pl.ds(s * ZROWS, ZROWS)])


def _tcmm_body(x_ref, w_ref, h_ref):
    h_ref[...] = jnp.dot(x_ref[...], w_ref[...],
                         preferred_element_type=jnp.float32)


def _tcmm(x, w1):
    return pl.pallas_call(
        _tcmm_body,
        out_shape=jax.ShapeDtypeStruct((N, H), jnp.float32),
    )(x, w1)


def _tc1_body(h_ref, d0_ref, d1_ref, hp_ref, dinv_ref):
    deg = d0_ref[...] + d1_ref[...] + 1.0
    dinv = lax.rsqrt(deg)
    dinv_ref[...] = dinv
    hp_ref[...] = h_ref[...] * dinv


def _tc1(h1, d0, d1):
    return pl.pallas_call(
        _tc1_body,
        out_shape=[jax.ShapeDtypeStruct((N, H), jnp.float32),
                   jax.ShapeDtypeStruct((N, 1), jnp.float32)],
    )(h1, d0, d1)


def _tcmid_body(a0_ref, a1_ref, hp_ref, dinv_ref, b_ref, g_ref, be_ref,
                wn_ref, out_ref):
    dinv = dinv_ref[...]
    z = jnp.maximum(dinv * (a0_ref[...] + a1_ref[...] + hp_ref[...])
                    + b_ref[...], 0.0)
    m = jnp.mean(z, axis=0, keepdims=True)
    v = jnp.mean((z - m) ** 2, axis=0, keepdims=True)
    zn = (z - m) * lax.rsqrt(v + 1e-5) * g_ref[...] + be_ref[...]
    out_ref[...] = jnp.dot(zn, wn_ref[...],
                           preferred_element_type=jnp.float32) * dinv


def _tcmid(a0, a1, hp, dinv, b, g, be, wn):
    return pl.pallas_call(
        _tcmid_body,
        out_shape=jax.ShapeDtypeStruct((N, H), jnp.float32),
    )(a0, a1, hp, dinv, b, g, be, wn)


def _tcfin_body(a0_ref, a1_ref, hp_ref, dinv_ref, b_ref, g_ref, be_ref,
                wc_ref, bc_ref, out_ref):
    dinv = dinv_ref[...]
    z = jnp.maximum(dinv * (a0_ref[...] + a1_ref[...] + hp_ref[...])
                    + b_ref[...], 0.0)
    m = jnp.mean(z, axis=0, keepdims=True)
    v = jnp.mean((z - m) ** 2, axis=0, keepdims=True)
    zn = (z - m) * lax.rsqrt(v + 1e-5) * g_ref[...] + be_ref[...]
    out_ref[...] = jnp.dot(zn, wc_ref[...],
                           preferred_element_type=jnp.float32) + bc_ref[...]


def _tcfin(a0, a1, hp, dinv, b, g, be, wc, bc):
    return pl.pallas_call(
        _tcfin_body,
        out_shape=jax.ShapeDtypeStruct((N, 2), jnp.float32),
    )(a0, a1, hp, dinv, b, g, be, wc, bc)


def kernel(x, edge_index, W1, b1, W2, b2, W3, b3,
           g1, be1, g2, be2, g3, be3, Wc, bc):
    src = edge_index[0]
    dst = edge_index[1]
    padlen = EPAD - E
    srcb = jnp.concatenate(
        [src, jnp.zeros((padlen,), src.dtype)]).reshape(NW * CPT, CHUNK)
    pad_dst = N + jnp.arange(padlen, dtype=dst.dtype) % (R - N)
    dstb = jnp.concatenate([dst, pad_dst]).reshape(NW * CPT, CHUNK)
    srcb = jnp.pad(srcb, ((0, 32), (0, 0)))
    dstb = jnp.pad(dstb, ((0, 32), (0, 0)))
    ones_deg = jnp.ones((CHUNK, H), jnp.float32)

    h1 = _tcmm(x, W1)
    degp = _sc_degree(dstb, ones_deg)
    d0 = degp[0, :N, 0:1]
    d1 = degp[1, :N, 0:1]
    hp, dinv = _tc1(h1, d0, d1)

    layer_params = [(b1, g1, be1, W2), (b2, g2, be2, W3)]
    for b, g, be, wn in layer_params:
        agg = _sc_scatter(hp, srcb, dstb)
        hp = _tcmid(agg[0, :N], agg[1, :N], hp, dinv,
                    b.reshape(1, H), g.reshape(1, H), be.reshape(1, H), wn)

    agg = _sc_scatter(hp, srcb, dstb)
    out = _tcfin(agg[0, :N], agg[1, :N], hp, dinv,
                 b3.reshape(1, H), g3.reshape(1, H), be3.reshape(1, H),
                 Wc, bc.reshape(1, 2))
    return out
